# baseline (speedup 1.0000x reference)
import functools

import jax
import jax.numpy as jnp
from jax.experimental import pallas as pl
from jax.experimental.pallas import tpu as pltpu

N = 10000
D_IN = 128
D_HID = 128
N_CLASS = 16
ROW_BLK = 400
NB = N // ROW_BLK


def _fused_kernel(adj_ref, x_ref, w1_ref, b1_ref, w2_ref, b2_ref,
                  out_ref, yself_ref, yneigh_ref):
    i = pl.program_id(0)
    adj = adj_ref[...]
    deg = jnp.sum(adj, axis=1, keepdims=True)
    deg = jnp.maximum(deg, 1e-12)
    adj16 = adj.astype(jnp.bfloat16)
    blk = jax.lax.rem(i, NB)
    row = blk * ROW_BLK

    @pl.when(i < NB)
    def _layer1():
        acc = jax.lax.dot_general(
            adj16, x_ref[...],
            (((1,), (0,)), ((), ())), preferred_element_type=jnp.float32)
        neigh = acc / deg
        w1 = w1_ref[...]
        xs = x_ref[pl.ds(row, ROW_BLK), :].astype(jnp.float32)
        h = (jax.lax.dot_general(xs, w1[:D_IN],
                                 (((1,), (0,)), ((), ())),
                                 preferred_element_type=jnp.float32)
             + jax.lax.dot_general(neigh, w1[D_IN:],
                                   (((1,), (0,)), ((), ())),
                                   preferred_element_type=jnp.float32)
             + b1_ref[...])
        h = jnp.maximum(h, 0.0)
        w2 = w2_ref[...]
        yself_ref[pl.ds(row, ROW_BLK), :] = jax.lax.dot_general(
            h, w2[:D_HID], (((1,), (0,)), ((), ())),
            preferred_element_type=jnp.float32) + b2_ref[...]
        yneigh_ref[pl.ds(row, ROW_BLK), :] = jax.lax.dot_general(
            h, w2[D_HID:], (((1,), (0,)), ((), ())),
            preferred_element_type=jnp.float32)

    @pl.when(i >= NB)
    def _layer2():
        acc = jax.lax.dot_general(
            adj16, yneigh_ref[...].astype(jnp.bfloat16),
            (((1,), (0,)), ((), ())), preferred_element_type=jnp.float32)
        logits = yself_ref[pl.ds(row, ROW_BLK), :] + acc / deg
        m = jnp.max(logits, axis=1, keepdims=True)
        s = logits - m
        lse = jnp.log(jnp.sum(jnp.exp(s), axis=1, keepdims=True))
        out_ref[...] = s - lse


@functools.partial(jax.jit, static_argnames=("interpret",))
def kernel(feature, adj, W1, b1, W2, b2, interpret=False):
    b1r = b1.reshape(1, D_HID)
    b2r = b2.reshape(1, N_CLASS)
    x16 = feature.astype(jnp.bfloat16)

    out = pl.pallas_call(
        _fused_kernel,
        grid=(2 * NB,),
        in_specs=[
            pl.BlockSpec((ROW_BLK, N), lambda i: (jax.lax.rem(i, NB), 0)),
            pl.BlockSpec((N, D_IN), lambda i: (0, 0)),
            pl.BlockSpec((2 * D_IN, D_HID), lambda i: (0, 0)),
            pl.BlockSpec((1, D_HID), lambda i: (0, 0)),
            pl.BlockSpec((2 * D_HID, N_CLASS), lambda i: (0, 0)),
            pl.BlockSpec((1, N_CLASS), lambda i: (0, 0)),
        ],
        out_specs=pl.BlockSpec((ROW_BLK, N_CLASS),
                               lambda i: (jax.lax.rem(i, NB), 0)),
        out_shape=jax.ShapeDtypeStruct((N, N_CLASS), jnp.float32),
        scratch_shapes=[
            pltpu.VMEM((N, N_CLASS), jnp.float32),
            pltpu.VMEM((N, N_CLASS), jnp.float32),
        ],
        interpret=interpret,
    )(adj, x16, W1, b1r, W2, b2r)
    return out


# speedup vs baseline: 1.0557x; 1.0557x over previous
import functools

import jax
import jax.numpy as jnp
from jax.experimental import pallas as pl
from jax.experimental.pallas import tpu as pltpu

N = 10000
D_IN = 128
D_HID = 128
N_CLASS = 16
ROW_BLK = 400
NB = N // ROW_BLK


def _fused_kernel(adj_ref, x_ref, w1_ref, b1_ref, w2_ref, b2_ref,
                  out_ref, ys_ref, yneigh_ref):
    i = pl.program_id(0)
    blk = jax.lax.rem(i, NB)
    row = blk * ROW_BLK

    @pl.when(i < NB)
    def _layer1():
        adj = adj_ref[...]
        rdeg = 1.0 / jnp.maximum(jnp.sum(adj, axis=1, keepdims=True), 1e-12)
        acc = jax.lax.dot_general(
            adj.astype(jnp.bfloat16), x_ref[...],
            (((1,), (0,)), ((), ())), preferred_element_type=jnp.float32)
        neigh = acc * rdeg
        w1 = w1_ref[...]
        xs = x_ref[pl.ds(row, ROW_BLK), :].astype(jnp.float32)
        h = (jax.lax.dot_general(xs, w1[:D_IN],
                                 (((1,), (0,)), ((), ())),
                                 preferred_element_type=jnp.float32)
             + jax.lax.dot_general(neigh, w1[D_IN:],
                                   (((1,), (0,)), ((), ())),
                                   preferred_element_type=jnp.float32)
             + b1_ref[...])
        h = jnp.maximum(h, 0.0)
        w2 = w2_ref[...]
        ys_ref[pl.ds(row, ROW_BLK), :N_CLASS] = jax.lax.dot_general(
            h, w2[:D_HID], (((1,), (0,)), ((), ())),
            preferred_element_type=jnp.float32) + b2_ref[...]
        ys_ref[pl.ds(row, ROW_BLK), N_CLASS:] = jnp.broadcast_to(
            rdeg, (ROW_BLK, N_CLASS))
        yneigh_ref[pl.ds(row, ROW_BLK), :] = jax.lax.dot_general(
            h, w2[D_HID:], (((1,), (0,)), ((), ())),
            preferred_element_type=jnp.float32).astype(jnp.bfloat16)

    @pl.when(i >= NB)
    def _layer2():
        acc = jax.lax.dot_general(
            adj_ref[...].astype(jnp.bfloat16), yneigh_ref[...],
            (((1,), (0,)), ((), ())), preferred_element_type=jnp.float32)
        yb = ys_ref[pl.ds(row, ROW_BLK), :]
        logits = yb[:, :N_CLASS] + acc * yb[:, N_CLASS:]
        m = jnp.max(logits, axis=1, keepdims=True)
        s = logits - m
        lse = jnp.log(jnp.sum(jnp.exp(s), axis=1, keepdims=True))
        out_ref[...] = s - lse


@functools.partial(jax.jit, static_argnames=("interpret",))
def kernel(feature, adj, W1, b1, W2, b2, interpret=False):
    b1r = b1.reshape(1, D_HID)
    b2r = b2.reshape(1, N_CLASS)
    x16 = feature.astype(jnp.bfloat16)

    out = pl.pallas_call(
        _fused_kernel,
        grid=(2 * NB,),
        in_specs=[
            pl.BlockSpec((ROW_BLK, N), lambda i: (jax.lax.rem(i, NB), 0)),
            pl.BlockSpec((N, D_IN), lambda i: (0, 0)),
            pl.BlockSpec((2 * D_IN, D_HID), lambda i: (0, 0)),
            pl.BlockSpec((1, D_HID), lambda i: (0, 0)),
            pl.BlockSpec((2 * D_HID, N_CLASS), lambda i: (0, 0)),
            pl.BlockSpec((1, N_CLASS), lambda i: (0, 0)),
        ],
        out_specs=pl.BlockSpec((ROW_BLK, N_CLASS),
                               lambda i: (jax.lax.rem(i, NB), 0)),
        out_shape=jax.ShapeDtypeStruct((N, N_CLASS), jnp.float32),
        scratch_shapes=[
            pltpu.VMEM((N, 2 * N_CLASS), jnp.float32),
            pltpu.VMEM((N, N_CLASS), jnp.bfloat16),
        ],
        interpret=interpret,
    )(adj, x16, W1, b1r, W2, b2r)
    return out
